# baseline (speedup 1.0000x reference)
import functools

import jax
import jax.numpy as jnp
from jax import lax
from jax.experimental import pallas as pl
from jax.experimental.pallas import tpu as pltpu
from jax.experimental.pallas import tpu_sc as plsc

B = 16384
D = 64
NC = 2
NS = 16
NW = NC * NS
BPW = B // NW
CHUNK = 128
NCHUNK = BPW // CHUNK


@functools.cache
def _build():
    mesh = plsc.VectorSubcoreMesh(core_axis_name="c", subcore_axis_name="s")

    @functools.partial(
        pl.kernel,
        out_type=jax.ShapeDtypeStruct((B, D), jnp.float32),
        mesh=mesh,
        scratch_types=[
            pltpu.VMEM((NCHUNK, CHUNK), jnp.int32),
            pltpu.VMEM((NCHUNK, CHUNK), jnp.int32),
            pltpu.VMEM((BPW, D), jnp.float32),
            pltpu.VMEM((BPW, D), jnp.float32),
            pltpu.SemaphoreType.DMA,
            pltpu.SemaphoreType.DMA,
        ],
        compiler_params=pltpu.CompilerParams(use_tc_tiling_on_sc=False),
    )
    def _emb_prod(idx0_hbm, idx1_hbm, emb_in_hbm, emb_out_hbm, out_hbm,
                  idx0_v, idx1_v, rows_a, rows_b, sem_a, sem_b):
        wid = lax.axis_index("s") * NC + lax.axis_index("c")
        base = wid * BPW

        pltpu.sync_copy(idx0_hbm.at[wid], idx0_v.at[...])
        pltpu.sync_copy(idx1_hbm.at[wid], idx1_v.at[...])

        copies = []
        for j in range(NCHUNK):
            copies.append(pltpu.async_copy(
                emb_in_hbm.at[idx0_v.at[j]],
                rows_a.at[pl.ds(j * CHUNK, CHUNK), :], sem_a))
            copies.append(pltpu.async_copy(
                emb_out_hbm.at[idx1_v.at[j]],
                rows_b.at[pl.ds(j * CHUNK, CHUNK), :], sem_b))
        for c in copies:
            c.wait()

        def body(r, carry):
            for k in range(D // 16):
                a = rows_a[r, pl.ds(k * 16, 16)]
                b = rows_b[r, pl.ds(k * 16, 16)]
                rows_a[r, pl.ds(k * 16, 16)] = a * b
            return carry
        lax.fori_loop(0, BPW, body, 0)

        pltpu.sync_copy(rows_a.at[...], out_hbm.at[pl.ds(base, BPW), :])

    return _emb_prod


def kernel(g, emb_in, emb_out):
    g = g.astype(jnp.int32)
    idx0 = g[:, 0].reshape(NW, NCHUNK, CHUNK)
    idx1 = g[:, 1].reshape(NW, NCHUNK, CHUNK)
    return _build()(idx0, idx1, emb_in, emb_out)


# speedup vs baseline: 2.7119x; 2.7119x over previous
import functools

import jax
import jax.numpy as jnp
from jax import lax
from jax.experimental import pallas as pl
from jax.experimental.pallas import tpu as pltpu
from jax.experimental.pallas import tpu_sc as plsc

V = 1000000
D = 64
B = 16384
NS = 16
BPT = B // NS
DPC = D // 2


@functools.cache
def _build():
    mesh = plsc.VectorSubcoreMesh(core_axis_name="c", subcore_axis_name="s")

    @functools.partial(
        pl.kernel,
        out_type=jax.ShapeDtypeStruct((D, B), jnp.float32),
        mesh=mesh,
        scratch_types=[
            pltpu.VMEM((BPT,), jnp.int32),
            pltpu.VMEM((BPT,), jnp.int32),
            pltpu.VMEM((BPT,), jnp.float32),
            pltpu.VMEM((BPT,), jnp.float32),
            pltpu.VMEM_SHARED((V,), jnp.float32),
            pltpu.VMEM_SHARED((V,), jnp.float32),
            pltpu.SemaphoreType.DMA,
            pltpu.SemaphoreType.DMA,
            pltpu.SemaphoreType.DMA,
            pltpu.SemaphoreType.DMA,
        ],
    )
    def _emb_prod(idx0_hbm, idx1_hbm, at_hbm, bt_hbm, out_hbm,
                  idx0_v, idx1_v, aval_v, bval_v, shA, shB,
                  semA, semB, gsemA, gsemB):
        cid = lax.axis_index("c")
        sid = lax.axis_index("s")
        bbase = sid * BPT
        pltpu.sync_copy(idx0_hbm.at[sid], idx0_v)
        pltpu.sync_copy(idx1_hbm.at[sid], idx1_v)

        def body(i, carry):
            d = cid * DPC + i

            @pl.when(sid == 0)
            def _():
                ca = pltpu.async_copy(at_hbm.at[d], shA, semA)
                cb = pltpu.async_copy(bt_hbm.at[d], shB, semB)
                ca.wait()
                cb.wait()

            plsc.subcore_barrier()
            ga = pltpu.async_copy(shA.at[idx0_v], aval_v, gsemA)
            gb = pltpu.async_copy(shB.at[idx1_v], bval_v, gsemB)
            ga.wait()
            gb.wait()

            def mbody(k, c2):
                a = aval_v[pl.ds(k * 16, 16)]
                b = bval_v[pl.ds(k * 16, 16)]
                aval_v[pl.ds(k * 16, 16)] = a * b
                return c2

            lax.fori_loop(0, BPT // 16, mbody, 0)
            pltpu.sync_copy(aval_v, out_hbm.at[d, pl.ds(bbase, BPT)])
            plsc.subcore_barrier()
            return carry

        lax.fori_loop(0, DPC, body, 0)

    return _emb_prod


def kernel(g, emb_in, emb_out):
    g = g.astype(jnp.int32)
    idx0 = g[:, 0].reshape(NS, BPT)
    idx1 = g[:, 1].reshape(NS, BPT)
    out_p = _build()(idx0, idx1, emb_in.T, emb_out.T)
    return out_p.T
